# baseline (speedup 1.0000x reference)
import functools

import jax
import jax.numpy as jnp
from jax import lax
from jax.experimental import pallas as pl
from jax.experimental.pallas import tpu as pltpu
from jax.experimental.pallas import tpu_sc as plsc

N = 10000
D_IN = 128
HID = 256
EMB = 128
OUT = 64
E_RAW = 320000
E_TOT = E_RAW + N

NSUB = 16
K = 96
SB = 16
NSC = 14
CHUNKS = SB * NSC
EPC = CHUNKS * K
E_PAD = EPC * NSUB
N_PAD = 10240
ROWS = N_PAD // 128
HALF = N_PAD // 2
HROWS = HALF // 128

K2 = 64
SB2 = 8
CAP = 14336

BR = 200
NB = N // BR


def _prep0_body(x_ref, w_ref, as_ref, ad_ref,
                h_ref, es_ref, ed_ref, gmax_ref, gm_s):
    i = pl.program_id(0)
    h = jnp.dot(x_ref[...], w_ref[...], preferred_element_type=jnp.float32)
    es = jnp.sum(h * as_ref[...], axis=1, keepdims=True)
    ed = jnp.sum(h * ad_ref[...], axis=1, keepdims=True)
    h_ref[...] = h.reshape(2 * BR, 128)
    es_ref[...] = es
    ed_ref[...] = ed
    bmax = jnp.max(es)

    @pl.when(i == 0)
    def _():
        gm_s[0, 0] = bmax

    @pl.when(i > 0)
    def _():
        gm_s[0, 0] = jnp.maximum(gm_s[0, 0], bmax)

    @pl.when(i == NB - 1)
    def _():
        gmax_ref[...] = jnp.full((8, 128), gm_s[0, 0], jnp.float32)


def _prep0(x, W, a_s, a_d):
    return pl.pallas_call(
        _prep0_body,
        grid=(NB,),
        in_specs=[
            pl.BlockSpec((BR, D_IN), lambda i: (i, 0)),
            pl.BlockSpec((D_IN, HID), lambda i: (0, 0)),
            pl.BlockSpec((1, HID), lambda i: (0, 0)),
            pl.BlockSpec((1, HID), lambda i: (0, 0)),
        ],
        out_specs=[
            pl.BlockSpec((2 * BR, 128), lambda i: (i, 0)),
            pl.BlockSpec((BR, 1), lambda i: (i, 0)),
            pl.BlockSpec((BR, 1), lambda i: (i, 0)),
            pl.BlockSpec((8, 128), lambda i: (0, 0)),
        ],
        out_shape=[
            jax.ShapeDtypeStruct((2 * N, 128), jnp.float32),
            jax.ShapeDtypeStruct((N, 1), jnp.float32),
            jax.ShapeDtypeStruct((N, 1), jnp.float32),
            jax.ShapeDtypeStruct((8, 128), jnp.float32),
        ],
        scratch_shapes=[pltpu.SMEM((1, 1), jnp.float32)],
    )(x, W, a_s, a_d)


def _mid_body(t_ref, b_ref, g_ref, be_ref, w_ref, as_ref, ad_ref,
              h_ref, es_ref, ed_ref, gmax_ref,
              ssum, ssq, gm_s):
    p = pl.program_id(0)
    i = pl.program_id(1)
    t = t_ref[...] + b_ref[...]

    @pl.when(p == 0)
    def _():
        s1 = jnp.sum(t, axis=0, keepdims=True)
        s2 = jnp.sum(t * t, axis=0, keepdims=True)

        @pl.when(i == 0)
        def _():
            ssum[...] = s1
            ssq[...] = s2

        @pl.when(i > 0)
        def _():
            ssum[...] = ssum[...] + s1
            ssq[...] = ssq[...] + s2

    @pl.when(p == 1)
    def _():
        mu = ssum[...] * (1.0 / N)
        var = ssq[...] * (1.0 / N) - mu * mu
        y = (t - mu) * jax.lax.rsqrt(var + 1e-5) * g_ref[...] + be_ref[...]
        y = jnp.maximum(y, 0.0)
        h = jnp.dot(y, w_ref[...], preferred_element_type=jnp.float32)
        es = jnp.sum(h * as_ref[...], axis=1, keepdims=True)
        ed = jnp.sum(h * ad_ref[...], axis=1, keepdims=True)
        h_ref[...] = h.reshape(2 * BR, 128)
        es_ref[...] = es
        ed_ref[...] = ed
        bmax = jnp.max(es)

        @pl.when(i == 0)
        def _():
            gm_s[0, 0] = bmax

        @pl.when(i > 0)
        def _():
            gm_s[0, 0] = jnp.maximum(gm_s[0, 0], bmax)

        @pl.when(i == NB - 1)
        def _():
            gmax_ref[...] = jnp.full((8, 128), gm_s[0, 0], jnp.float32)


def _mid(agg, b, g, be, W, a_s, a_d):
    return pl.pallas_call(
        _mid_body,
        grid=(2, NB),
        in_specs=[
            pl.BlockSpec((BR, HID), lambda p, i: (i, 0)),
            pl.BlockSpec((1, HID), lambda p, i: (0, 0)),
            pl.BlockSpec((1, HID), lambda p, i: (0, 0)),
            pl.BlockSpec((1, HID), lambda p, i: (0, 0)),
            pl.BlockSpec((HID, HID), lambda p, i: (0, 0)),
            pl.BlockSpec((1, HID), lambda p, i: (0, 0)),
            pl.BlockSpec((1, HID), lambda p, i: (0, 0)),
        ],
        out_specs=[
            pl.BlockSpec((2 * BR, 128), lambda p, i: (i, 0)),
            pl.BlockSpec((BR, 1), lambda p, i: (i, 0)),
            pl.BlockSpec((BR, 1), lambda p, i: (i, 0)),
            pl.BlockSpec((8, 128), lambda p, i: (0, 0)),
        ],
        out_shape=[
            jax.ShapeDtypeStruct((2 * N, 128), jnp.float32),
            jax.ShapeDtypeStruct((N, 1), jnp.float32),
            jax.ShapeDtypeStruct((N, 1), jnp.float32),
            jax.ShapeDtypeStruct((8, 128), jnp.float32),
        ],
        scratch_shapes=[
            pltpu.VMEM((1, HID), jnp.float32),
            pltpu.VMEM((1, HID), jnp.float32),
            pltpu.SMEM((1, 1), jnp.float32),
        ],
    )(agg, b, g, be, W, a_s, a_d)


def _final_body(t_ref, b_ref, wt_ref, bt_ref, g_ref, be_ref,
                wl0_ref, bl0_ref, wl1_ref, bl1_ref, out_ref, ssum, ssq):
    p = pl.program_id(0)
    i = pl.program_id(1)
    t = t_ref[...] + b_ref[...]
    u = jnp.dot(t, wt_ref[...], preferred_element_type=jnp.float32) + bt_ref[...]

    @pl.when(p == 0)
    def _():
        s1 = jnp.sum(u, axis=0, keepdims=True)
        s2 = jnp.sum(u * u, axis=0, keepdims=True)

        @pl.when(i == 0)
        def _():
            ssum[...] = s1
            ssq[...] = s2

        @pl.when(i > 0)
        def _():
            ssum[...] = ssum[...] + s1
            ssq[...] = ssq[...] + s2

    @pl.when(p == 1)
    def _():
        mu = ssum[...] * (1.0 / N)
        var = ssq[...] * (1.0 / N) - mu * mu
        y = (u - mu) * jax.lax.rsqrt(var + 1e-5) * g_ref[...] + be_ref[...]
        y = jnp.maximum(y, 0.0)
        z = jnp.dot(y, wl0_ref[...], preferred_element_type=jnp.float32) + bl0_ref[...]
        z = jnp.maximum(z, 0.0)
        out_ref[...] = jnp.dot(z, wl1_ref[...], preferred_element_type=jnp.float32) + bl1_ref[...]


def _final(agg, b, Wt, bt, g, be, Wl0, bl0, Wl1, bl1):
    HE = HID + EMB
    return pl.pallas_call(
        _final_body,
        grid=(2, NB),
        in_specs=[
            pl.BlockSpec((BR, HID), lambda p, i: (i, 0)),
            pl.BlockSpec((1, HID), lambda p, i: (0, 0)),
            pl.BlockSpec((HID, HE), lambda p, i: (0, 0)),
            pl.BlockSpec((1, HE), lambda p, i: (0, 0)),
            pl.BlockSpec((1, HE), lambda p, i: (0, 0)),
            pl.BlockSpec((1, HE), lambda p, i: (0, 0)),
            pl.BlockSpec((HE, HID), lambda p, i: (0, 0)),
            pl.BlockSpec((1, HID), lambda p, i: (0, 0)),
            pl.BlockSpec((HID, OUT), lambda p, i: (0, 0)),
            pl.BlockSpec((1, OUT), lambda p, i: (0, 0)),
        ],
        out_specs=pl.BlockSpec((BR, OUT), lambda p, i: (i, 0)),
        out_shape=jax.ShapeDtypeStruct((N, OUT), jnp.float32),
        scratch_shapes=[
            pltpu.VMEM((1, HE), jnp.float32),
            pltpu.VMEM((1, HE), jnp.float32),
        ],
    )(agg, b, Wt, bt, g, be, Wl0, bl0, Wl1, bl1)


def _sc_p1_body(es_hbm, ed_hbm, gm_hbm, src_hbm, dst_hbm,
                csrc_hbm, cdst_hbm, cex_hbm, cnt_hbm, denp_hbm,
                es_v, ed_v, den_v, srcb, dstb, csrc_v, cdst_v, cex_v,
                idx_v, gm_v, cnt_v, den_sh):
    cid = lax.axis_index("c")
    sid = lax.axis_index("s")
    pltpu.sync_copy(es_hbm, es_v.at[pl.ds(0, N)])
    pltpu.sync_copy(ed_hbm, ed_v.at[pl.ds(0, N)])
    pltpu.sync_copy(gm_hbm, gm_v)

    zeros16 = jnp.zeros((16,), jnp.float32)
    zeros16i = jnp.zeros((16,), jnp.int32)

    def _zrow(i, _):
        def _zcol(j, _):
            den_v[i, pl.ds(j * 16, 16)] = zeros16
            return ()
        lax.fori_loop(0, 128 // 16, _zcol, ())
        return ()
    lax.fori_loop(0, ROWS, _zrow, ())

    def _zcomp(i, _):
        csrc_v[pl.ds(i * 16, 16)] = zeros16i
        cdst_v[pl.ds(i * 16, 16)] = zeros16i
        cex_v[pl.ds(i * 16, 16)] = zeros16
        return ()
    lax.fori_loop(0, CAP // 16, _zcomp, ())

    @pl.when(sid == 0)
    def _():
        pltpu.sync_copy(den_v, den_sh)

    i16 = lax.iota(jnp.int32, 16)

    def _zi(i, _):
        idx_v[pl.ds(i * 16, 16)] = i * 16 + i16
        return ()
    lax.fori_loop(0, ROWS // 16, _zi, ())
    plsc.subcore_barrier()

    gm = gm_v[...]
    nlo = cid * HALF

    def _superchunk(scg, off):
        pltpu.sync_copy(src_hbm.at[sid, pl.ds(scg * SB, SB)], srcb)
        pltpu.sync_copy(dst_hbm.at[sid, pl.ds(scg * SB, SB)], dstb)

        def _chunk(j, off):
            def _vec(t, off):
                s_idx = srcb[j, pl.ds(t * 16, 16)]
                d_idx = dstb[j, pl.ds(t * 16, 16)]
                es_g = plsc.load_gather(es_v, [s_idx])
                ed_g = plsc.load_gather(ed_v, [d_idx])
                s = es_g + ed_g
                e = jnp.maximum(s, 0.2 * s)
                c = jnp.maximum(0.0, gm + ed_g)
                ex = jnp.exp(e - c)
                eid = sid * EPC + (scg * SB + j) * K + t * 16 + i16
                valid = eid < E_TOT
                ex = jnp.where(valid, ex, 0.0)
                plsc.addupdate_scatter(
                    den_v,
                    [jnp.right_shift(d_idx, 7), jnp.bitwise_and(d_idx, 127)],
                    ex)
                d_loc = d_idx - nlo
                m = valid & (d_loc >= 0) & (d_loc < HALF) & (off < CAP - 16)
                plsc.store_compressed(csrc_v.at[pl.ds(off, 16)], s_idx, mask=m)
                plsc.store_compressed(cdst_v.at[pl.ds(off, 16)], d_loc, mask=m)
                plsc.store_compressed(cex_v.at[pl.ds(off, 16)], ex, mask=m)
                pc = plsc.all_reduce_population_count(m)
                return off + lax.reduce_max(pc, (0,))
            return lax.fori_loop(0, K // 16, _vec, off)
        return lax.fori_loop(0, SB, _chunk, off)
    count = lax.fori_loop(0, NSC, _superchunk, jnp.int32(0))

    pltpu.sync_copy(csrc_v, csrc_hbm.at[cid, pl.ds(sid * CAP, CAP)])
    pltpu.sync_copy(cdst_v, cdst_hbm.at[cid, pl.ds(sid * CAP, CAP)])
    pltpu.sync_copy(cex_v, cex_hbm.at[cid, pl.ds(sid * CAP, CAP)])
    cnt_splat = zeros16i + count

    def _wcnt(i, _):
        cnt_v[pl.ds(i * 16, 16)] = cnt_splat
        return ()
    lax.fori_loop(0, 128 // 16, _wcnt, ())
    pltpu.sync_copy(cnt_v, cnt_hbm.at[cid, pl.ds(sid * 128, 128)])

    pltpu.sync_copy(den_v, den_sh.at[idx_v], add=True)
    plsc.subcore_barrier()

    @pl.when(sid < ROWS // 8)
    def _():
        pltpu.sync_copy(den_sh.at[pl.ds(sid * 8, 8), :],
                        denp_hbm.at[cid, pl.ds(sid * 8, 8), :])


@functools.lru_cache(maxsize=1)
def _build_sc_p1():
    return pl.kernel(
        _sc_p1_body,
        out_type=[
            jax.ShapeDtypeStruct((2, NSUB * CAP), jnp.int32),
            jax.ShapeDtypeStruct((2, NSUB * CAP), jnp.int32),
            jax.ShapeDtypeStruct((2, NSUB * CAP), jnp.float32),
            jax.ShapeDtypeStruct((2, NSUB * 128), jnp.int32),
            jax.ShapeDtypeStruct((2, ROWS, 128), jnp.float32),
        ],
        mesh=plsc.VectorSubcoreMesh(core_axis_name="c", subcore_axis_name="s",
                                    num_cores=2, num_subcores=NSUB),
        compiler_params=pltpu.CompilerParams(needs_layout_passes=False),
        scratch_types=[
            pltpu.VMEM((N_PAD,), jnp.float32),
            pltpu.VMEM((N_PAD,), jnp.float32),
            pltpu.VMEM((ROWS, 128), jnp.float32),
            pltpu.VMEM((SB, K), jnp.int32),
            pltpu.VMEM((SB, K), jnp.int32),
            pltpu.VMEM((CAP,), jnp.int32),
            pltpu.VMEM((CAP,), jnp.int32),
            pltpu.VMEM((CAP,), jnp.float32),
            pltpu.VMEM((ROWS,), jnp.int32),
            pltpu.VMEM((16,), jnp.float32),
            pltpu.VMEM((128,), jnp.int32),
            pltpu.VMEM_SHARED((ROWS, 128), jnp.float32),
        ],
    )


def _sc_p2_body(csrc_hbm, cdst_hbm, cex_hbm, cnt_hbm, denp_hbm, h_hbm,
                agg0_hbm, agg1_hbm,
                r_v, rows_a, rows_b, src1, dst1, ex1, src2, dst2, alpha2,
                cnt_v, acc_sh, gsem_a, gsem_b, ssem_a, ssem_b):
    cid = lax.axis_index("c")
    sid = lax.axis_index("s")
    i16 = lax.iota(jnp.int32, 16)

    def _core(agg_hbm):
        pltpu.sync_copy(denp_hbm.at[0, pl.ds(cid * HROWS, HROWS), :], r_v)

        def _rrow(i, _):
            def _rcol(j, _):
                r_v[i, pl.ds(j * 16, 16)] = (
                    1.0 / (r_v[i, pl.ds(j * 16, 16)] + 1e-16))
                return ()
            lax.fori_loop(0, 128 // 16, _rcol, ())
            return ()
        lax.fori_loop(0, HROWS, _rrow, ())

        zeros16 = jnp.zeros((16,), jnp.float32)

        def _zrow(i, _):
            def _zcol(j, _):
                rows_a[i, pl.ds(j * 16, 16)] = zeros16
                return ()
            lax.fori_loop(0, 128 // 16, _zcol, ())
            return ()
        lax.fori_loop(0, 2 * K2, _zrow, ())
        for t in range(2 * HALF // NSUB // (2 * K2)):
            pltpu.sync_copy(
                rows_a,
                acc_sh.at[pl.ds(sid * (2 * HALF // NSUB) + t * 2 * K2,
                                2 * K2), :])
        plsc.subcore_barrier()

        pltpu.sync_copy(cnt_hbm.at[cid, pl.ds(sid * 128, 128)], cnt_v)
        n = lax.reduce_max(cnt_v[pl.ds(0, 16)], (0,))
        trips = jnp.right_shift(n + (SB2 * K2 - 1), 9)

        def _wait_gather(rows, gsem):
            pltpu.make_async_copy(
                h_hbm.at[src2.at[0]], rows, gsem).wait()

        def _wait_scatter(rows, ssem):
            pltpu.make_async_copy(rows, acc_sh.at[dst2.at[0]], ssem).wait()

        def _process(j, rows, gsem, ssem):
            _wait_gather(rows, gsem)

            def _avec(t, _):
                d_idx = dst1[pl.ds(j * K2 + t * 16, 16)]
                ex = ex1[pl.ds(j * K2 + t * 16, 16)]
                r_g = plsc.load_gather(
                    r_v,
                    [jnp.right_shift(d_idx, 7), jnp.bitwise_and(d_idx, 127)])
                a = ex * r_g
                pos = t * 32 + 2 * i16
                plsc.store_scatter(alpha2, [pos], a)
                plsc.store_scatter(alpha2, [pos + 1], a)
                return ()
            lax.fori_loop(0, K2 // 16, _avec, ())

            def _scale(q, _):
                for u in range(4):
                    kk = q * 4 + u
                    a = plsc.load_gather(
                        alpha2, [jnp.full((16,), 0, jnp.int32) + kk])
                    for f in range(128 // 16):
                        rows[kk, pl.ds(f * 16, 16)] = (
                            rows[kk, pl.ds(f * 16, 16)] * a)
                return ()
            lax.fori_loop(0, 2 * K2 // 4, _scale, ())

            pltpu.async_copy(rows, acc_sh.at[dst2.at[j]], ssem, add=True)

        def _sct(sct, _):
            base = sid * CAP + sct * (SB2 * K2)
            pltpu.sync_copy(csrc_hbm.at[cid, pl.ds(base, SB2 * K2)], src1)
            pltpu.sync_copy(cdst_hbm.at[cid, pl.ds(base, SB2 * K2)], dst1)
            pltpu.sync_copy(cex_hbm.at[cid, pl.ds(base, SB2 * K2)], ex1)

            def _dbl(j, _):
                def _dblv(t, _):
                    s = src1[pl.ds(j * K2 + t * 16, 16)]
                    d = dst1[pl.ds(j * K2 + t * 16, 16)]
                    js = jnp.full((16,), 0, jnp.int32) + j
                    pos = t * 32 + 2 * i16
                    plsc.store_scatter(src2, [js, pos], 2 * s)
                    plsc.store_scatter(src2, [js, pos + 1], 2 * s + 1)
                    plsc.store_scatter(dst2, [js, pos], 2 * d)
                    plsc.store_scatter(dst2, [js, pos + 1], 2 * d + 1)
                    return ()
                lax.fori_loop(0, K2 // 16, _dblv, ())
                return ()
            lax.fori_loop(0, SB2, _dbl, ())

            pltpu.async_copy(h_hbm.at[src2.at[0]], rows_a, gsem_a)

            def _pair(q, _):
                j0 = 2 * q

                @pl.when(q > 0)
                def _():
                    _wait_scatter(rows_b, ssem_b)
                pltpu.async_copy(h_hbm.at[src2.at[j0 + 1]], rows_b, gsem_b)
                _process(j0, rows_a, gsem_a, ssem_a)

                @pl.when(q < SB2 // 2 - 1)
                def _():
                    _wait_scatter(rows_a, ssem_a)
                    pltpu.async_copy(h_hbm.at[src2.at[j0 + 2]], rows_a, gsem_a)
                _process(j0 + 1, rows_b, gsem_b, ssem_b)
                return ()
            lax.fori_loop(0, SB2 // 2, _pair, ())
            _wait_scatter(rows_a, ssem_a)
            _wait_scatter(rows_b, ssem_b)
            return ()
        lax.fori_loop(0, trips, _sct, ())
        plsc.subcore_barrier()

        for t in range(2 * HALF // NSUB // (2 * K2)):
            pltpu.sync_copy(
                acc_sh.at[pl.ds(sid * (2 * HALF // NSUB) + t * 2 * K2,
                                2 * K2), :],
                rows_a)
            pltpu.sync_copy(
                rows_a,
                agg_hbm.at[pl.ds(sid * (2 * HALF // NSUB) + t * 2 * K2,
                                 2 * K2), :])

    @pl.when(cid == 0)
    def _():
        _core(agg0_hbm)

    @pl.when(cid == 1)
    def _():
        _core(agg1_hbm)


@functools.lru_cache(maxsize=1)
def _build_sc_p2():
    return pl.kernel(
        _sc_p2_body,
        out_type=[
            jax.ShapeDtypeStruct((2 * HALF, 128), jnp.float32),
            jax.ShapeDtypeStruct((2 * HALF, 128), jnp.float32),
        ],
        mesh=plsc.VectorSubcoreMesh(core_axis_name="c", subcore_axis_name="s",
                                    num_cores=2, num_subcores=NSUB),
        compiler_params=pltpu.CompilerParams(needs_layout_passes=False),
        scratch_types=[
            pltpu.VMEM((HROWS, 128), jnp.float32),
            pltpu.VMEM((2 * K2, 128), jnp.float32),
            pltpu.VMEM((2 * K2, 128), jnp.float32),
            pltpu.VMEM((SB2 * K2,), jnp.int32),
            pltpu.VMEM((SB2 * K2,), jnp.int32),
            pltpu.VMEM((SB2 * K2,), jnp.float32),
            pltpu.VMEM((SB2, 2 * K2), jnp.int32),
            pltpu.VMEM((SB2, 2 * K2), jnp.int32),
            pltpu.VMEM((2 * K2,), jnp.float32),
            pltpu.VMEM((128,), jnp.int32),
            pltpu.VMEM_SHARED((2 * HALF, 128), jnp.float32),
            pltpu.SemaphoreType.DMA,
            pltpu.SemaphoreType.DMA,
            pltpu.SemaphoreType.DMA,
            pltpu.SemaphoreType.DMA,
        ],
    )


def kernel(x, edge_index, W0, as0, ad0, b0, W1, as1, ad1, b1, W2, as2, ad2, b2,
           g0, be0, g1, be1, g2, be2, Wt, bt, Wl0, bl0, Wl1, bl1):
    loop = jnp.arange(N, dtype=edge_index.dtype)
    src = jnp.concatenate([edge_index[0], loop])
    dst = jnp.concatenate([edge_index[1], loop])
    pad = E_PAD - E_TOT
    src = jnp.pad(src, (0, pad)).reshape(NSUB, CHUNKS, K)
    dst = jnp.pad(dst, (0, pad)).reshape(NSUB, CHUNKS, K)

    r2 = lambda v: v.reshape(1, -1)
    sc_p1 = _build_sc_p1()
    sc_p2 = _build_sc_p2()

    def sc_layer(h, es, ed, gmax):
        csrc, cdst, cex, cnt, denp = sc_p1(
            es.reshape(-1), ed.reshape(-1), gmax.reshape(-1)[:16], src, dst)
        a0, a1 = sc_p2(csrc, cdst, cex, cnt, denp, h)
        return jnp.concatenate([a0.reshape(HALF, HID),
                                a1.reshape(HALF, HID)], axis=0)

    h, es, ed, gmax = _prep0(x, W0, r2(as0), r2(ad0))
    agg = sc_layer(h, es, ed, gmax)
    h, es, ed, gmax = _mid(agg, r2(b0), r2(g0), r2(be0), W1, r2(as1), r2(ad1))
    agg = sc_layer(h, es, ed, gmax)
    h, es, ed, gmax = _mid(agg, r2(b1), r2(g1), r2(be1), W2, r2(as2), r2(ad2))
    agg = sc_layer(h, es, ed, gmax)
    return _final(agg, r2(b2), Wt, r2(bt), r2(g2), r2(be2),
                  Wl0, r2(bl0), Wl1, r2(bl1))


# speedup vs baseline: 1.1364x; 1.1364x over previous
import functools

import jax
import jax.numpy as jnp
from jax import lax
from jax.experimental import pallas as pl
from jax.experimental.pallas import tpu as pltpu
from jax.experimental.pallas import tpu_sc as plsc

N = 10000
D_IN = 128
HID = 256
EMB = 128
OUT = 64
E_RAW = 320000
E_TOT = E_RAW + N

NSUB = 16
K = 96
SB = 16
NSC = 14
CHUNKS = SB * NSC
EPC = CHUNKS * K
E_PAD = EPC * NSUB
N_PAD = 10240
ROWS = N_PAD // 128
HALF = N_PAD // 2
HROWS = HALF // 128

K2 = 48
SB2 = 8
CAP = 13824

BR = 200
NB = N // BR


def _prep0_body(x_ref, w_ref, as_ref, ad_ref,
                h_ref, es_ref, ed_ref, gmax_ref, gm_s):
    i = pl.program_id(0)
    h = jnp.dot(x_ref[...], w_ref[...], preferred_element_type=jnp.float32)
    es = jnp.sum(h * as_ref[...], axis=1, keepdims=True)
    ed = jnp.sum(h * ad_ref[...], axis=1, keepdims=True)
    h_ref[...] = h.reshape(2 * BR, 128)
    es_ref[...] = es
    ed_ref[...] = ed
    bmax = jnp.max(es)

    @pl.when(i == 0)
    def _():
        gm_s[0, 0] = bmax

    @pl.when(i > 0)
    def _():
        gm_s[0, 0] = jnp.maximum(gm_s[0, 0], bmax)

    @pl.when(i == NB - 1)
    def _():
        gmax_ref[...] = jnp.full((8, 128), gm_s[0, 0], jnp.float32)


def _prep0(x, W, a_s, a_d):
    return pl.pallas_call(
        _prep0_body,
        grid=(NB,),
        in_specs=[
            pl.BlockSpec((BR, D_IN), lambda i: (i, 0)),
            pl.BlockSpec((D_IN, HID), lambda i: (0, 0)),
            pl.BlockSpec((1, HID), lambda i: (0, 0)),
            pl.BlockSpec((1, HID), lambda i: (0, 0)),
        ],
        out_specs=[
            pl.BlockSpec((2 * BR, 128), lambda i: (i, 0)),
            pl.BlockSpec((BR, 1), lambda i: (i, 0)),
            pl.BlockSpec((BR, 1), lambda i: (i, 0)),
            pl.BlockSpec((8, 128), lambda i: (0, 0)),
        ],
        out_shape=[
            jax.ShapeDtypeStruct((2 * N, 128), jnp.float32),
            jax.ShapeDtypeStruct((N, 1), jnp.float32),
            jax.ShapeDtypeStruct((N, 1), jnp.float32),
            jax.ShapeDtypeStruct((8, 128), jnp.float32),
        ],
        scratch_shapes=[pltpu.SMEM((1, 1), jnp.float32)],
    )(x, W, a_s, a_d)


def _mid_body(t_ref, b_ref, g_ref, be_ref, w_ref, as_ref, ad_ref,
              h_ref, es_ref, ed_ref, gmax_ref,
              ssum, ssq, gm_s):
    p = pl.program_id(0)
    i = pl.program_id(1)
    t = t_ref[...] + b_ref[...]

    @pl.when(p == 0)
    def _():
        s1 = jnp.sum(t, axis=0, keepdims=True)
        s2 = jnp.sum(t * t, axis=0, keepdims=True)

        @pl.when(i == 0)
        def _():
            ssum[...] = s1
            ssq[...] = s2

        @pl.when(i > 0)
        def _():
            ssum[...] = ssum[...] + s1
            ssq[...] = ssq[...] + s2

    @pl.when(p == 1)
    def _():
        mu = ssum[...] * (1.0 / N)
        var = ssq[...] * (1.0 / N) - mu * mu
        y = (t - mu) * jax.lax.rsqrt(var + 1e-5) * g_ref[...] + be_ref[...]
        y = jnp.maximum(y, 0.0)
        h = jnp.dot(y, w_ref[...], preferred_element_type=jnp.float32)
        es = jnp.sum(h * as_ref[...], axis=1, keepdims=True)
        ed = jnp.sum(h * ad_ref[...], axis=1, keepdims=True)
        h_ref[...] = h.reshape(2 * BR, 128)
        es_ref[...] = es
        ed_ref[...] = ed
        bmax = jnp.max(es)

        @pl.when(i == 0)
        def _():
            gm_s[0, 0] = bmax

        @pl.when(i > 0)
        def _():
            gm_s[0, 0] = jnp.maximum(gm_s[0, 0], bmax)

        @pl.when(i == NB - 1)
        def _():
            gmax_ref[...] = jnp.full((8, 128), gm_s[0, 0], jnp.float32)


def _mid(agg, b, g, be, W, a_s, a_d):
    return pl.pallas_call(
        _mid_body,
        grid=(2, NB),
        in_specs=[
            pl.BlockSpec((BR, HID), lambda p, i: (i, 0)),
            pl.BlockSpec((1, HID), lambda p, i: (0, 0)),
            pl.BlockSpec((1, HID), lambda p, i: (0, 0)),
            pl.BlockSpec((1, HID), lambda p, i: (0, 0)),
            pl.BlockSpec((HID, HID), lambda p, i: (0, 0)),
            pl.BlockSpec((1, HID), lambda p, i: (0, 0)),
            pl.BlockSpec((1, HID), lambda p, i: (0, 0)),
        ],
        out_specs=[
            pl.BlockSpec((2 * BR, 128), lambda p, i: (i, 0)),
            pl.BlockSpec((BR, 1), lambda p, i: (i, 0)),
            pl.BlockSpec((BR, 1), lambda p, i: (i, 0)),
            pl.BlockSpec((8, 128), lambda p, i: (0, 0)),
        ],
        out_shape=[
            jax.ShapeDtypeStruct((2 * N, 128), jnp.float32),
            jax.ShapeDtypeStruct((N, 1), jnp.float32),
            jax.ShapeDtypeStruct((N, 1), jnp.float32),
            jax.ShapeDtypeStruct((8, 128), jnp.float32),
        ],
        scratch_shapes=[
            pltpu.VMEM((1, HID), jnp.float32),
            pltpu.VMEM((1, HID), jnp.float32),
            pltpu.SMEM((1, 1), jnp.float32),
        ],
    )(agg, b, g, be, W, a_s, a_d)


def _final_body(t_ref, b_ref, wt_ref, bt_ref, g_ref, be_ref,
                wl0_ref, bl0_ref, wl1_ref, bl1_ref, out_ref, ssum, ssq):
    p = pl.program_id(0)
    i = pl.program_id(1)
    t = t_ref[...] + b_ref[...]
    u = jnp.dot(t, wt_ref[...], preferred_element_type=jnp.float32) + bt_ref[...]

    @pl.when(p == 0)
    def _():
        s1 = jnp.sum(u, axis=0, keepdims=True)
        s2 = jnp.sum(u * u, axis=0, keepdims=True)

        @pl.when(i == 0)
        def _():
            ssum[...] = s1
            ssq[...] = s2

        @pl.when(i > 0)
        def _():
            ssum[...] = ssum[...] + s1
            ssq[...] = ssq[...] + s2

    @pl.when(p == 1)
    def _():
        mu = ssum[...] * (1.0 / N)
        var = ssq[...] * (1.0 / N) - mu * mu
        y = (u - mu) * jax.lax.rsqrt(var + 1e-5) * g_ref[...] + be_ref[...]
        y = jnp.maximum(y, 0.0)
        z = jnp.dot(y, wl0_ref[...], preferred_element_type=jnp.float32) + bl0_ref[...]
        z = jnp.maximum(z, 0.0)
        out_ref[...] = jnp.dot(z, wl1_ref[...], preferred_element_type=jnp.float32) + bl1_ref[...]


def _final(agg, b, Wt, bt, g, be, Wl0, bl0, Wl1, bl1):
    HE = HID + EMB
    return pl.pallas_call(
        _final_body,
        grid=(2, NB),
        in_specs=[
            pl.BlockSpec((BR, HID), lambda p, i: (i, 0)),
            pl.BlockSpec((1, HID), lambda p, i: (0, 0)),
            pl.BlockSpec((HID, HE), lambda p, i: (0, 0)),
            pl.BlockSpec((1, HE), lambda p, i: (0, 0)),
            pl.BlockSpec((1, HE), lambda p, i: (0, 0)),
            pl.BlockSpec((1, HE), lambda p, i: (0, 0)),
            pl.BlockSpec((HE, HID), lambda p, i: (0, 0)),
            pl.BlockSpec((1, HID), lambda p, i: (0, 0)),
            pl.BlockSpec((HID, OUT), lambda p, i: (0, 0)),
            pl.BlockSpec((1, OUT), lambda p, i: (0, 0)),
        ],
        out_specs=pl.BlockSpec((BR, OUT), lambda p, i: (i, 0)),
        out_shape=jax.ShapeDtypeStruct((N, OUT), jnp.float32),
        scratch_shapes=[
            pltpu.VMEM((1, HE), jnp.float32),
            pltpu.VMEM((1, HE), jnp.float32),
        ],
    )(agg, b, Wt, bt, g, be, Wl0, bl0, Wl1, bl1)


def _sc_p1_body(es_hbm, ed_hbm, gm_hbm, src_hbm, dst_hbm,
                csrc_hbm, cdst_hbm, cex_hbm, cnt_hbm, denp_hbm,
                es_v, ed_v, den_v, srcb, dstb, csrc_v, cdst_v, cex_v,
                idx_v, gm_v, cnt_v, den_sh):
    cid = lax.axis_index("c")
    sid = lax.axis_index("s")
    pltpu.sync_copy(es_hbm, es_v.at[pl.ds(0, N)])
    pltpu.sync_copy(ed_hbm, ed_v.at[pl.ds(0, N)])
    pltpu.sync_copy(gm_hbm, gm_v)

    zeros16 = jnp.zeros((16,), jnp.float32)
    zeros16i = jnp.zeros((16,), jnp.int32)

    def _zrow(i, _):
        def _zcol(j, _):
            den_v[i, pl.ds(j * 16, 16)] = zeros16
            return ()
        lax.fori_loop(0, 128 // 16, _zcol, ())
        return ()
    lax.fori_loop(0, ROWS, _zrow, ())

    def _zcomp(i, _):
        csrc_v[pl.ds(i * 16, 16)] = zeros16i
        cdst_v[pl.ds(i * 16, 16)] = zeros16i
        cex_v[pl.ds(i * 16, 16)] = zeros16
        return ()
    lax.fori_loop(0, CAP // 16, _zcomp, ())

    @pl.when(sid == 0)
    def _():
        pltpu.sync_copy(den_v, den_sh)

    i16 = lax.iota(jnp.int32, 16)

    def _zi(i, _):
        idx_v[pl.ds(i * 16, 16)] = i * 16 + i16
        return ()
    lax.fori_loop(0, ROWS // 16, _zi, ())
    plsc.subcore_barrier()

    gm = gm_v[...]
    nlo = cid * HALF

    def _superchunk(scg, off):
        pltpu.sync_copy(src_hbm.at[sid, pl.ds(scg * SB, SB)], srcb)
        pltpu.sync_copy(dst_hbm.at[sid, pl.ds(scg * SB, SB)], dstb)

        def _chunk(j, off):
            def _vec(t, off):
                s_idx = srcb[j, pl.ds(t * 16, 16)]
                d_idx = dstb[j, pl.ds(t * 16, 16)]
                es_g = plsc.load_gather(es_v, [s_idx])
                ed_g = plsc.load_gather(ed_v, [d_idx])
                s = es_g + ed_g
                e = jnp.maximum(s, 0.2 * s)
                c = jnp.maximum(0.0, gm + ed_g)
                ex = jnp.exp(e - c)
                eid = sid * EPC + (scg * SB + j) * K + t * 16 + i16
                valid = eid < E_TOT
                ex = jnp.where(valid, ex, 0.0)
                plsc.addupdate_scatter(
                    den_v,
                    [jnp.right_shift(d_idx, 7), jnp.bitwise_and(d_idx, 127)],
                    ex)
                d_loc = d_idx - nlo
                m = valid & (d_loc >= 0) & (d_loc < HALF) & (off < CAP - 16)
                plsc.store_compressed(csrc_v.at[pl.ds(off, 16)], s_idx, mask=m)
                plsc.store_compressed(cdst_v.at[pl.ds(off, 16)], d_loc, mask=m)
                plsc.store_compressed(cex_v.at[pl.ds(off, 16)], ex, mask=m)
                pc = plsc.all_reduce_population_count(m)
                return off + lax.reduce_max(pc, (0,))
            return lax.fori_loop(0, K // 16, _vec, off)
        return lax.fori_loop(0, SB, _chunk, off)
    count = lax.fori_loop(0, NSC, _superchunk, jnp.int32(0))

    pltpu.sync_copy(csrc_v, csrc_hbm.at[cid, pl.ds(sid * CAP, CAP)])
    pltpu.sync_copy(cdst_v, cdst_hbm.at[cid, pl.ds(sid * CAP, CAP)])
    pltpu.sync_copy(cex_v, cex_hbm.at[cid, pl.ds(sid * CAP, CAP)])
    cnt_splat = zeros16i + count

    def _wcnt(i, _):
        cnt_v[pl.ds(i * 16, 16)] = cnt_splat
        return ()
    lax.fori_loop(0, 128 // 16, _wcnt, ())
    pltpu.sync_copy(cnt_v, cnt_hbm.at[cid, pl.ds(sid * 128, 128)])

    pltpu.sync_copy(den_v, den_sh.at[idx_v], add=True)
    plsc.subcore_barrier()

    @pl.when(sid < ROWS // 8)
    def _():
        pltpu.sync_copy(den_sh.at[pl.ds(sid * 8, 8), :],
                        denp_hbm.at[cid, pl.ds(sid * 8, 8), :])


@functools.lru_cache(maxsize=1)
def _build_sc_p1():
    return pl.kernel(
        _sc_p1_body,
        out_type=[
            jax.ShapeDtypeStruct((2, NSUB * CAP), jnp.int32),
            jax.ShapeDtypeStruct((2, NSUB * CAP), jnp.int32),
            jax.ShapeDtypeStruct((2, NSUB * CAP), jnp.float32),
            jax.ShapeDtypeStruct((2, NSUB * 128), jnp.int32),
            jax.ShapeDtypeStruct((2, ROWS, 128), jnp.float32),
        ],
        mesh=plsc.VectorSubcoreMesh(core_axis_name="c", subcore_axis_name="s",
                                    num_cores=2, num_subcores=NSUB),
        compiler_params=pltpu.CompilerParams(needs_layout_passes=False),
        scratch_types=[
            pltpu.VMEM((N_PAD,), jnp.float32),
            pltpu.VMEM((N_PAD,), jnp.float32),
            pltpu.VMEM((ROWS, 128), jnp.float32),
            pltpu.VMEM((SB, K), jnp.int32),
            pltpu.VMEM((SB, K), jnp.int32),
            pltpu.VMEM((CAP,), jnp.int32),
            pltpu.VMEM((CAP,), jnp.int32),
            pltpu.VMEM((CAP,), jnp.float32),
            pltpu.VMEM((ROWS,), jnp.int32),
            pltpu.VMEM((16,), jnp.float32),
            pltpu.VMEM((128,), jnp.int32),
            pltpu.VMEM_SHARED((ROWS, 128), jnp.float32),
        ],
    )


def _sc_p2_body(csrc_hbm, cdst_hbm, cex_hbm, cnt_hbm, denp_hbm, h_hbm,
                agg0_hbm, agg1_hbm,
                r_v, rows_a, rows_b, rows_c, src1, dst1, ex1, src2, dst2,
                alpha2, cnt_v, acc_sh,
                gsem_a, gsem_b, gsem_c, ssem_a, ssem_b, ssem_c):
    cid = lax.axis_index("c")
    sid = lax.axis_index("s")
    i16 = lax.iota(jnp.int32, 16)
    BATCH = SB2 * K2
    bufs = [(rows_a, gsem_a, ssem_a),
            (rows_b, gsem_b, ssem_b),
            (rows_c, gsem_c, ssem_c)]

    def _core(agg_hbm):
        pltpu.sync_copy(denp_hbm.at[0, pl.ds(cid * HROWS, HROWS), :], r_v)

        def _rrow(i, _):
            def _rcol(j, _):
                r_v[i, pl.ds(j * 16, 16)] = (
                    1.0 / (r_v[i, pl.ds(j * 16, 16)] + 1e-16))
                return ()
            lax.fori_loop(0, 128 // 16, _rcol, ())
            return ()
        lax.fori_loop(0, HROWS, _rrow, ())

        zeros16 = jnp.zeros((16,), jnp.float32)

        def _zrow(i, _):
            def _zcol(j, _):
                rows_a[i, pl.ds(j * 16, 16)] = zeros16
                return ()
            lax.fori_loop(0, 128 // 16, _zcol, ())
            return ()
        lax.fori_loop(0, 2 * K2, _zrow, ())
        rps = 2 * HALF // NSUB
        off = 0
        for sz in ([2 * K2] * (rps // (2 * K2))) + (
                [rps % (2 * K2)] if rps % (2 * K2) else []):
            pltpu.sync_copy(rows_a.at[pl.ds(0, sz), :],
                            acc_sh.at[pl.ds(sid * rps + off, sz), :])
            off += sz
        plsc.subcore_barrier()

        pltpu.sync_copy(cnt_hbm.at[cid, pl.ds(sid * 128, 128)], cnt_v)
        n = lax.reduce_max(cnt_v[pl.ds(0, 16)], (0,))
        trips = (n + (BATCH - 1)) // BATCH

        def _wait_gather(rows, gsem):
            pltpu.make_async_copy(h_hbm.at[src2.at[0]], rows, gsem).wait()

        def _wait_scatter(rows, ssem):
            pltpu.make_async_copy(rows, acc_sh.at[dst2.at[0]], ssem).wait()

        def _issue_gather(j, rows, gsem):
            pltpu.async_copy(h_hbm.at[src2.at[j]], rows, gsem)

        def _process(j, rows, gsem, ssem):
            _wait_gather(rows, gsem)

            def _avec(t, _):
                d_idx = dst1[pl.ds(j * K2 + t * 16, 16)]
                ex = ex1[pl.ds(j * K2 + t * 16, 16)]
                r_g = plsc.load_gather(
                    r_v,
                    [jnp.right_shift(d_idx, 7), jnp.bitwise_and(d_idx, 127)])
                a = ex * r_g
                pos = t * 32 + 2 * i16
                plsc.store_scatter(alpha2, [pos], a)
                plsc.store_scatter(alpha2, [pos + 1], a)
                return ()
            lax.fori_loop(0, K2 // 16, _avec, ())

            def _scale(q, _):
                for u in range(4):
                    kk = q * 4 + u
                    a = plsc.load_gather(
                        alpha2, [jnp.full((16,), 0, jnp.int32) + kk])
                    for f in range(128 // 16):
                        rows[kk, pl.ds(f * 16, 16)] = (
                            rows[kk, pl.ds(f * 16, 16)] * a)
                return ()
            lax.fori_loop(0, 2 * K2 // 4, _scale, ())

            pltpu.async_copy(rows, acc_sh.at[dst2.at[j]], ssem, add=True)

        def _sct(sct, _):
            base = sid * CAP + sct * BATCH
            pltpu.sync_copy(csrc_hbm.at[cid, pl.ds(base, BATCH)], src1)
            pltpu.sync_copy(cdst_hbm.at[cid, pl.ds(base, BATCH)], dst1)
            pltpu.sync_copy(cex_hbm.at[cid, pl.ds(base, BATCH)], ex1)

            def _dbl(j, _):
                def _dblv(t, _):
                    s = src1[pl.ds(j * K2 + t * 16, 16)]
                    d = dst1[pl.ds(j * K2 + t * 16, 16)]
                    js = jnp.full((16,), 0, jnp.int32) + j
                    pos = t * 32 + 2 * i16
                    plsc.store_scatter(src2, [js, pos], 2 * s)
                    plsc.store_scatter(src2, [js, pos + 1], 2 * s + 1)
                    plsc.store_scatter(dst2, [js, pos], 2 * d)
                    plsc.store_scatter(dst2, [js, pos + 1], 2 * d + 1)
                    return ()
                lax.fori_loop(0, K2 // 16, _dblv, ())
                return ()
            lax.fori_loop(0, SB2, _dbl, ())

            _issue_gather(0, *bufs[0][:1], bufs[0][1])
            _issue_gather(1, *bufs[1][:1], bufs[1][1])
            for j in range(SB2):
                rows, gsem, ssem = bufs[j % 3]
                _process(j, rows, gsem, ssem)
                if j + 2 < SB2:
                    prows, pgsem, pssem = bufs[(j + 2) % 3]
                    if j >= 1:
                        _wait_scatter(prows, pssem)
                    _issue_gather(j + 2, prows, pgsem)
            for rows, gsem, ssem in bufs:
                _wait_scatter(rows, ssem)
            return ()
        lax.fori_loop(0, trips, _sct, ())
        plsc.subcore_barrier()

        off = 0
        for sz in ([2 * K2] * (rps // (2 * K2))) + (
                [rps % (2 * K2)] if rps % (2 * K2) else []):
            pltpu.sync_copy(acc_sh.at[pl.ds(sid * rps + off, sz), :],
                            rows_a.at[pl.ds(0, sz), :])
            pltpu.sync_copy(rows_a.at[pl.ds(0, sz), :],
                            agg_hbm.at[pl.ds(sid * rps + off, sz), :])
            off += sz

    @pl.when(cid == 0)
    def _():
        _core(agg0_hbm)

    @pl.when(cid == 1)
    def _():
        _core(agg1_hbm)


@functools.lru_cache(maxsize=1)
def _build_sc_p2():
    return pl.kernel(
        _sc_p2_body,
        out_type=[
            jax.ShapeDtypeStruct((2 * HALF, 128), jnp.float32),
            jax.ShapeDtypeStruct((2 * HALF, 128), jnp.float32),
        ],
        mesh=plsc.VectorSubcoreMesh(core_axis_name="c", subcore_axis_name="s",
                                    num_cores=2, num_subcores=NSUB),
        compiler_params=pltpu.CompilerParams(needs_layout_passes=False),
        scratch_types=[
            pltpu.VMEM((HROWS, 128), jnp.float32),
            pltpu.VMEM((2 * K2, 128), jnp.float32),
            pltpu.VMEM((2 * K2, 128), jnp.float32),
            pltpu.VMEM((2 * K2, 128), jnp.float32),
            pltpu.VMEM((SB2 * K2,), jnp.int32),
            pltpu.VMEM((SB2 * K2,), jnp.int32),
            pltpu.VMEM((SB2 * K2,), jnp.float32),
            pltpu.VMEM((SB2, 2 * K2), jnp.int32),
            pltpu.VMEM((SB2, 2 * K2), jnp.int32),
            pltpu.VMEM((2 * K2,), jnp.float32),
            pltpu.VMEM((128,), jnp.int32),
            pltpu.VMEM_SHARED((2 * HALF, 128), jnp.float32),
            pltpu.SemaphoreType.DMA,
            pltpu.SemaphoreType.DMA,
            pltpu.SemaphoreType.DMA,
            pltpu.SemaphoreType.DMA,
            pltpu.SemaphoreType.DMA,
            pltpu.SemaphoreType.DMA,
        ],
    )


def kernel(x, edge_index, W0, as0, ad0, b0, W1, as1, ad1, b1, W2, as2, ad2, b2,
           g0, be0, g1, be1, g2, be2, Wt, bt, Wl0, bl0, Wl1, bl1):
    loop = jnp.arange(N, dtype=edge_index.dtype)
    src = jnp.concatenate([edge_index[0], loop])
    dst = jnp.concatenate([edge_index[1], loop])
    pad = E_PAD - E_TOT
    src = jnp.pad(src, (0, pad)).reshape(NSUB, CHUNKS, K)
    dst = jnp.pad(dst, (0, pad)).reshape(NSUB, CHUNKS, K)

    r2 = lambda v: v.reshape(1, -1)
    sc_p1 = _build_sc_p1()
    sc_p2 = _build_sc_p2()

    def sc_layer(h, es, ed, gmax):
        csrc, cdst, cex, cnt, denp = sc_p1(
            es.reshape(-1), ed.reshape(-1), gmax.reshape(-1)[:16], src, dst)
        a0, a1 = sc_p2(csrc, cdst, cex, cnt, denp, h)
        return jnp.concatenate([a0.reshape(HALF, HID),
                                a1.reshape(HALF, HID)], axis=0)

    h, es, ed, gmax = _prep0(x, W0, r2(as0), r2(ad0))
    agg = sc_layer(h, es, ed, gmax)
    h, es, ed, gmax = _mid(agg, r2(b0), r2(g0), r2(be0), W1, r2(as1), r2(ad1))
    agg = sc_layer(h, es, ed, gmax)
    h, es, ed, gmax = _mid(agg, r2(b1), r2(g1), r2(be1), W2, r2(as2), r2(ad2))
    agg = sc_layer(h, es, ed, gmax)
    return _final(agg, r2(b2), Wt, r2(bt), r2(g2), r2(be2),
                  Wl0, r2(bl0), Wl1, r2(bl1))
